# baseline (device time: 14236 ns/iter reference)
import jax
import jax.numpy as jnp
from jax import lax
from jax.experimental import pallas as pl
from jax.experimental.pallas import tpu as pltpu

B, Sq, Skv, Hq, Dh = 2, 128, 128, 16, 64
H_LOC = 4
D_HEADS = H_LOC * Dh
D_MODEL = 512
HALF = D_MODEL // 2
BLK = 64


def kernel(x, Wq, K_ext, V_ext, Wo):
    kf = K_ext.reshape(B, Skv, Hq * Dh)
    vf = V_ext.reshape(B, Skv, Hq * Dh)

    def body(x_ref, wq_ref, k_hbm, v_hbm, wo_ref, out_ref,
             k_vm, v_vm, cp_sems, send_ref, recv_ref, send_sems, recv_sems):
        my_i = lax.axis_index("i")
        p1 = my_i ^ 1
        p2 = 3 - my_i

        barrier = pltpu.get_barrier_semaphore()
        for p in (p1, p2):
            pl.semaphore_signal(
                barrier, inc=1,
                device_id=(p,), device_id_type=pl.DeviceIdType.MESH,
            )

        col0 = my_i * D_HEADS
        kv_copies = []
        for b in range(B):
            ck = pltpu.make_async_copy(
                k_hbm.at[b, :, pl.ds(col0, D_HEADS)], k_vm.at[b],
                cp_sems.at[0, b])
            cv = pltpu.make_async_copy(
                v_hbm.at[b, :, pl.ds(col0, D_HEADS)], v_vm.at[b],
                cp_sems.at[1, b])
            ck.start()
            cv.start()
            kv_copies.append((ck, cv))

        wqs = (wq_ref[...] * 0.125).astype(jnp.bfloat16)
        wo = wo_ref[...].astype(jnp.bfloat16)
        qs = [
            lax.dot(x_ref[b].astype(jnp.bfloat16), wqs,
                    preferred_element_type=jnp.float32).astype(jnp.bfloat16)
            for b in range(B)
        ]

        def exchange(step, b, half, p):
            lo = half * HALF
            r = pltpu.make_async_remote_copy(
                src_ref=send_ref.at[step, b, :, lo:lo + HALF],
                dst_ref=recv_ref.at[step, b, :, lo:lo + HALF],
                send_sem=send_sems.at[step, b, half],
                recv_sem=recv_sems.at[step, b, half],
                device_id=(p,),
                device_id_type=pl.DeviceIdType.MESH,
            )
            r.start()
            return r

        rdmas = []
        for b in range(B):
            q = qs[b]
            kv_copies[b][0].wait()
            kv_copies[b][1].wait()
            if b == 0:
                pl.semaphore_wait(barrier, 2)
            kb = k_vm[b].astype(jnp.bfloat16)
            vb = v_vm[b].astype(jnp.bfloat16)
            partial = None
            for h in range(H_LOC):
                qh = q[:, h * Dh:(h + 1) * Dh]
                kh = kb[:, h * Dh:(h + 1) * Dh]
                vh = vb[:, h * Dh:(h + 1) * Dh]
                s0 = lax.dot_general(
                    qh[0:BLK], kh[0:BLK], (((1,), (1,)), ((), ())),
                    preferred_element_type=jnp.float32)
                s1 = lax.dot_general(
                    qh[BLK:], kh, (((1,), (1,)), ((), ())),
                    preferred_element_type=jnp.float32)
                w0 = jnp.exp(s0)
                w1 = jnp.exp(s1)
                r0 = 1.0 / jnp.sum(w0, axis=-1, keepdims=True)
                r1 = 1.0 / jnp.sum(w1, axis=-1, keepdims=True)
                c0 = lax.dot(w0.astype(jnp.bfloat16), vh[0:BLK],
                             preferred_element_type=jnp.float32) * r0
                c1 = lax.dot(w1.astype(jnp.bfloat16), vh,
                             preferred_element_type=jnp.float32) * r1
                ctx = jnp.concatenate([c0, c1], axis=0).astype(jnp.bfloat16)
                d = lax.dot(ctx, wo[h * Dh:(h + 1) * Dh, :],
                            preferred_element_type=jnp.float32)
                partial = d if partial is None else partial + d
            out_ref[b] = partial
            send_ref[0, b] = partial.astype(jnp.bfloat16)
            rdmas.append(exchange(0, b, 0, p1))
            rdmas.append(exchange(0, b, 1, p2))

        for b in range(B):
            rdmas[2 * b].wait_recv()
            rdmas[2 * b + 1].wait_recv()
            acc = out_ref[b] + recv_ref[0, b].astype(jnp.float32)
            out_ref[b] = acc
            send_ref[1, b] = acc.astype(jnp.bfloat16)
            rdmas.append(exchange(1, b, 0, p2))
            rdmas.append(exchange(1, b, 1, p1))

        for b in range(B):
            rdmas[4 + 2 * b].wait_recv()
            rdmas[4 + 2 * b + 1].wait_recv()
            out_ref[b] = out_ref[b] + recv_ref[1, b].astype(jnp.float32)

        for r in rdmas:
            r.wait_send()

    return pl.pallas_call(
        body,
        out_shape=jax.ShapeDtypeStruct((B, Sq, D_MODEL), jnp.float32),
        in_specs=[
            pl.BlockSpec(memory_space=pltpu.VMEM),
            pl.BlockSpec(memory_space=pltpu.VMEM),
            pl.BlockSpec(memory_space=pltpu.MemorySpace.HBM),
            pl.BlockSpec(memory_space=pltpu.MemorySpace.HBM),
            pl.BlockSpec(memory_space=pltpu.VMEM),
        ],
        out_specs=pl.BlockSpec(memory_space=pltpu.VMEM),
        scratch_shapes=[
            pltpu.VMEM((B, Skv, D_HEADS), jnp.float32),
            pltpu.VMEM((B, Skv, D_HEADS), jnp.float32),
            pltpu.SemaphoreType.DMA((2, B)),
            pltpu.VMEM((2, B, Sq, D_MODEL), jnp.bfloat16),
            pltpu.VMEM((2, B, Sq, D_MODEL), jnp.bfloat16),
            pltpu.SemaphoreType.DMA((2, B, 2)),
            pltpu.SemaphoreType.DMA((2, B, 2)),
        ],
        compiler_params=pltpu.CompilerParams(collective_id=0),
    )(x, Wq, kf, vf, Wo)


# device time: 13604 ns/iter; 1.0465x vs baseline; 1.0465x over previous
import jax
import jax.numpy as jnp
from jax import lax
from jax.experimental import pallas as pl
from jax.experimental.pallas import tpu as pltpu

B, Sq, Skv, Hq, Dh = 2, 128, 128, 16, 64
H_LOC = 4
D_HEADS = H_LOC * Dh
D_MODEL = 512
HALF = D_MODEL // 2
BLK = 64


def kernel(x, Wq, K_ext, V_ext, Wo):
    my_pos = lax.axis_index("i")
    kf = lax.dynamic_slice_in_dim(
        K_ext.reshape(B, Skv, Hq * Dh), my_pos * D_HEADS, D_HEADS, axis=2
    ).astype(jnp.bfloat16)
    vf = lax.dynamic_slice_in_dim(
        V_ext.reshape(B, Skv, Hq * Dh), my_pos * D_HEADS, D_HEADS, axis=2
    ).astype(jnp.bfloat16)

    def body(x_ref, wq_ref, k_vm, v_vm, wo_ref, out_ref,
             send_ref, recv_ref, send_sems, recv_sems):
        my_i = lax.axis_index("i")
        p1 = my_i ^ 1
        p2 = 3 - my_i

        barrier = pltpu.get_barrier_semaphore()
        for p in (p1, p2):
            pl.semaphore_signal(
                barrier, inc=1,
                device_id=(p,), device_id_type=pl.DeviceIdType.MESH,
            )

        wqs = (wq_ref[...] * 0.125).astype(jnp.bfloat16)
        wo = wo_ref[...].astype(jnp.bfloat16)
        qs = [
            lax.dot(x_ref[b].astype(jnp.bfloat16), wqs,
                    preferred_element_type=jnp.float32).astype(jnp.bfloat16)
            for b in range(B)
        ]

        def exchange(step, b, half, p):
            lo = half * HALF
            r = pltpu.make_async_remote_copy(
                src_ref=send_ref.at[step, b, :, lo:lo + HALF],
                dst_ref=recv_ref.at[step, b, :, lo:lo + HALF],
                send_sem=send_sems.at[step, b, half],
                recv_sem=recv_sems.at[step, b, half],
                device_id=(p,),
                device_id_type=pl.DeviceIdType.MESH,
            )
            r.start()
            return r

        rdmas = []
        for b in range(B):
            q = qs[b]
            if b == 0:
                pl.semaphore_wait(barrier, 2)
            kb = k_vm[b]
            vb = v_vm[b]
            partial = None
            for h in range(H_LOC):
                qh = q[:, h * Dh:(h + 1) * Dh]
                kh = kb[:, h * Dh:(h + 1) * Dh]
                vh = vb[:, h * Dh:(h + 1) * Dh]
                s0 = lax.dot_general(
                    qh[0:BLK], kh[0:BLK], (((1,), (1,)), ((), ())),
                    preferred_element_type=jnp.float32)
                s1 = lax.dot_general(
                    qh[BLK:], kh, (((1,), (1,)), ((), ())),
                    preferred_element_type=jnp.float32)
                w0 = jnp.exp(s0)
                w1 = jnp.exp(s1)
                r0 = 1.0 / jnp.sum(w0, axis=-1, keepdims=True)
                r1 = 1.0 / jnp.sum(w1, axis=-1, keepdims=True)
                c0 = lax.dot(w0.astype(jnp.bfloat16), vh[0:BLK],
                             preferred_element_type=jnp.float32) * r0
                c1 = lax.dot(w1.astype(jnp.bfloat16), vh,
                             preferred_element_type=jnp.float32) * r1
                ctx = jnp.concatenate([c0, c1], axis=0).astype(jnp.bfloat16)
                d = lax.dot(ctx, wo[h * Dh:(h + 1) * Dh, :],
                            preferred_element_type=jnp.float32)
                partial = d if partial is None else partial + d
            out_ref[b] = partial
            send_ref[0, b] = partial.astype(jnp.bfloat16)
            rdmas.append(exchange(0, b, 0, p1))
            rdmas.append(exchange(0, b, 1, p2))

        for b in range(B):
            rdmas[2 * b].wait_recv()
            rdmas[2 * b + 1].wait_recv()
            acc = out_ref[b] + recv_ref[0, b].astype(jnp.float32)
            out_ref[b] = acc
            send_ref[1, b] = acc.astype(jnp.bfloat16)
            rdmas.append(exchange(1, b, 0, p2))
            rdmas.append(exchange(1, b, 1, p1))

        for b in range(B):
            rdmas[4 + 2 * b].wait_recv()
            rdmas[4 + 2 * b + 1].wait_recv()
            out_ref[b] = out_ref[b] + recv_ref[1, b].astype(jnp.float32)

        for r in rdmas:
            r.wait_send()

    return pl.pallas_call(
        body,
        out_shape=jax.ShapeDtypeStruct((B, Sq, D_MODEL), jnp.float32),
        in_specs=[
            pl.BlockSpec(memory_space=pltpu.VMEM),
            pl.BlockSpec(memory_space=pltpu.VMEM),
            pl.BlockSpec(memory_space=pltpu.VMEM),
            pl.BlockSpec(memory_space=pltpu.VMEM),
            pl.BlockSpec(memory_space=pltpu.VMEM),
        ],
        out_specs=pl.BlockSpec(memory_space=pltpu.VMEM),
        scratch_shapes=[
            pltpu.VMEM((2, B, Sq, D_MODEL), jnp.bfloat16),
            pltpu.VMEM((2, B, Sq, D_MODEL), jnp.bfloat16),
            pltpu.SemaphoreType.DMA((2, B, 2)),
            pltpu.SemaphoreType.DMA((2, B, 2)),
        ],
        compiler_params=pltpu.CompilerParams(collective_id=0),
    )(x, Wq, kf, vf, Wo)


# device time: 13395 ns/iter; 1.0628x vs baseline; 1.0156x over previous
import jax
import jax.numpy as jnp
from jax import lax
from jax.experimental import pallas as pl
from jax.experimental.pallas import tpu as pltpu

B, Sq, Skv, Hq, Dh = 2, 128, 128, 16, 64
H_LOC = 4
D_HEADS = H_LOC * Dh
D_MODEL = 512
HALF = D_MODEL // 2
BLK = 64


def kernel(x, Wq, K_ext, V_ext, Wo):
    my_pos = lax.axis_index("i")
    kf = lax.dynamic_slice_in_dim(
        K_ext.reshape(B, Skv, Hq * Dh), my_pos * D_HEADS, D_HEADS, axis=2)
    vf = lax.dynamic_slice_in_dim(
        V_ext.reshape(B, Skv, Hq * Dh), my_pos * D_HEADS, D_HEADS, axis=2)

    def body(x_ref, wq_ref, k_vm, v_vm, wo_ref, out_ref,
             acc_ref, send_ref, recv_ref, send_sems, recv_sems):
        my_i = lax.axis_index("i")
        p1 = my_i ^ 1
        p2 = 3 - my_i

        barrier = pltpu.get_barrier_semaphore()
        for p in (p1, p2):
            pl.semaphore_signal(
                barrier, inc=1,
                device_id=(p,), device_id_type=pl.DeviceIdType.MESH,
            )

        wqs = (wq_ref[...] * 0.125).astype(jnp.bfloat16)
        wo = wo_ref[...].astype(jnp.bfloat16)
        qs = [
            lax.dot(x_ref[b].astype(jnp.bfloat16), wqs,
                    preferred_element_type=jnp.float32).astype(jnp.bfloat16)
            for b in range(B)
        ]

        def exchange(step, b, half, p):
            lo = half * HALF
            r = pltpu.make_async_remote_copy(
                src_ref=send_ref.at[step, b, :, lo:lo + HALF],
                dst_ref=recv_ref.at[step, b, :, lo:lo + HALF],
                send_sem=send_sems.at[step, b, half],
                recv_sem=recv_sems.at[step, b, half],
                device_id=(p,),
                device_id_type=pl.DeviceIdType.MESH,
            )
            r.start()
            return r

        rdmas = []
        for b in range(B):
            q = qs[b]
            if b == 0:
                pl.semaphore_wait(barrier, 2)
            kb = k_vm[b].astype(jnp.bfloat16)
            vb = v_vm[b].astype(jnp.bfloat16)
            partial = None
            for h in range(H_LOC):
                qh = q[:, h * Dh:(h + 1) * Dh]
                kh = kb[:, h * Dh:(h + 1) * Dh]
                vh = vb[:, h * Dh:(h + 1) * Dh]
                s0 = lax.dot_general(
                    qh[0:BLK], kh[0:BLK], (((1,), (1,)), ((), ())),
                    preferred_element_type=jnp.float32)
                s1 = lax.dot_general(
                    qh[BLK:], kh, (((1,), (1,)), ((), ())),
                    preferred_element_type=jnp.float32)
                w0 = jnp.exp(s0)
                w1 = jnp.exp(s1)
                r0 = 1.0 / jnp.sum(w0, axis=-1, keepdims=True)
                r1 = 1.0 / jnp.sum(w1, axis=-1, keepdims=True)
                c0 = lax.dot(w0.astype(jnp.bfloat16), vh[0:BLK],
                             preferred_element_type=jnp.float32) * r0
                c1 = lax.dot(w1.astype(jnp.bfloat16), vh,
                             preferred_element_type=jnp.float32) * r1
                ctx = jnp.concatenate([c0, c1], axis=0).astype(jnp.bfloat16)
                d = lax.dot(ctx, wo[h * Dh:(h + 1) * Dh, :],
                            preferred_element_type=jnp.float32)
                partial = d if partial is None else partial + d
            acc_ref[b] = partial
            send_ref[0, b] = partial.astype(jnp.bfloat16)
            rdmas.append(exchange(0, b, 0, p1))
            rdmas.append(exchange(0, b, 1, p2))

        for b in range(B):
            rdmas[2 * b].wait_recv()
            rdmas[2 * b + 1].wait_recv()
            acc = acc_ref[b] + recv_ref[0, b].astype(jnp.float32)
            acc_ref[b] = acc
            send_ref[1, b] = acc.astype(jnp.bfloat16)
            rdmas.append(exchange(1, b, 0, p2))
            rdmas.append(exchange(1, b, 1, p1))

        for b in range(B):
            rdmas[4 + 2 * b].wait_recv()
            rdmas[4 + 2 * b + 1].wait_recv()
            out_ref[b] = (
                acc_ref[b] + recv_ref[1, b].astype(jnp.float32)
            ).astype(jnp.bfloat16)

        for r in rdmas:
            r.wait_send()

    return pl.pallas_call(
        body,
        out_shape=jax.ShapeDtypeStruct((B, Sq, D_MODEL), jnp.bfloat16),
        in_specs=[
            pl.BlockSpec(memory_space=pltpu.VMEM),
            pl.BlockSpec(memory_space=pltpu.VMEM),
            pl.BlockSpec(memory_space=pltpu.VMEM),
            pl.BlockSpec(memory_space=pltpu.VMEM),
            pl.BlockSpec(memory_space=pltpu.VMEM),
        ],
        out_specs=pl.BlockSpec(memory_space=pltpu.VMEM),
        scratch_shapes=[
            pltpu.VMEM((B, Sq, D_MODEL), jnp.float32),
            pltpu.VMEM((2, B, Sq, D_MODEL), jnp.bfloat16),
            pltpu.VMEM((2, B, Sq, D_MODEL), jnp.bfloat16),
            pltpu.SemaphoreType.DMA((2, B, 2)),
            pltpu.SemaphoreType.DMA((2, B, 2)),
        ],
        compiler_params=pltpu.CompilerParams(collective_id=0),
    )(x, Wq, kf, vf, Wo)
